# baseline (speedup 1.0000x reference)
import functools

import jax
import jax.numpy as jnp
from jax import lax
from jax.experimental import pallas as pl
from jax.experimental.pallas import tpu as pltpu
from jax.experimental.pallas import tpu_sc as plsc

D = 128
LANES = 16
NC = 2
NS = 16
NW = NC * NS
CH = 128
NSLOT = 3


def _lookup_add(x2, y2, w):
    n, _ = x2.shape
    hist, bsz = y2.shape
    n_chunks = hist
    b_w = bsz // NW
    mesh = plsc.VectorSubcoreMesh(core_axis_name="c", subcore_axis_name="s")

    idx_template = (
        jnp.arange(NSLOT * CH, dtype=jnp.int32).reshape(NSLOT, CH)
    )

    scratch = (
        [pltpu.VMEM((hist, b_w), jnp.int32)]
        + [pltpu.VMEM((NSLOT, CH), jnp.int32)]
        + [pltpu.VMEM((NSLOT, CH), jnp.int32)]
        + [pltpu.VMEM((CH, D), jnp.float32) for _ in range(NSLOT)]
        + [pltpu.VMEM((CH, D), jnp.float32)]
        + [pltpu.VMEM_SHARED((NS * NSLOT * CH, D), jnp.float32)]
        + [pltpu.SemaphoreType.DMA for _ in range(4 * NSLOT)]
    )

    @functools.partial(
        pl.kernel,
        mesh=mesh,
        out_type=jax.ShapeDtypeStruct(x2.shape, jnp.float32),
        scratch_types=scratch,
    )
    def k(x_hbm, y_hbm, w_hbm, idxt_hbm, out_hbm, idx_all, idxt, idxa,
          *bufs_and_sems):
        rows = bufs_and_sems[:NSLOT]
        dummy = bufs_and_sems[NSLOT]
        shared = bufs_and_sems[NSLOT + 1]
        gsem = bufs_and_sems[NSLOT + 2:2 * NSLOT + 2]
        xsem = bufs_and_sems[2 * NSLOT + 2:3 * NSLOT + 2]
        asem = bufs_and_sems[3 * NSLOT + 2:4 * NSLOT + 2]
        osem = bufs_and_sems[4 * NSLOT + 2:]

        sid = lax.axis_index("s")
        wid = sid * NC + lax.axis_index("c")
        b_base = wid * b_w
        s_base = sid * (NSLOT * CH)

        pltpu.sync_copy(y_hbm.at[:, pl.ds(b_base, b_w)], idx_all)
        pltpu.sync_copy(idxt_hbm, idxt)
        for sl in range(NSLOT):
            for t in range(CH // LANES):
                dl = pl.ds(t * LANES, LANES)
                idxa[sl, dl] = idxt[sl, dl] + s_base

        def issue_gather(c):
            s = c % NSLOT
            return pltpu.async_copy(
                w_hbm.at[idx_all.at[c]],
                rows[s],
                gsem[s],
            )

        def issue_x(c):
            s = c % NSLOT
            return pltpu.async_copy(
                x_hbm.at[pl.ds(c * bsz + b_base, CH)],
                shared.at[pl.ds(s_base + s * CH, CH)],
                xsem[s],
            )

        def issue_add(c):
            s = c % NSLOT
            return pltpu.async_copy(
                rows[s], shared.at[idxa.at[s]], asem[s], add=True
            )

        def issue_out(c):
            s = c % NSLOT
            return pltpu.async_copy(
                shared.at[pl.ds(s_base + s * CH, CH)],
                out_hbm.at[pl.ds(c * bsz + b_base, CH)],
                osem[s],
            )

        h_g = [None] * NSLOT
        h_x = [None] * NSLOT
        h_add = [None] * NSLOT
        h_out = [None] * NSLOT
        for c in range(min(NSLOT, n_chunks)):
            h_g[c] = issue_gather(c)
            h_x[c] = issue_x(c)

        for c in range(n_chunks):
            s = c % NSLOT
            h_g[s].wait()
            h_x[s].wait()
            h_add[s] = issue_add(c)
            rows_c = rows[s]

            @plsc.parallel_loop(0, CH, 1)
            def probe_body(ir):
                for t in range(D // LANES):
                    sl = pl.ds(t * LANES, LANES)
                    plsc.addupdate(dummy.at[ir, sl], rows_c[ir, sl])

            if c >= 1:
                sp = (c - 1) % NSLOT
                h_add[sp].wait()
                h_add[sp] = None
                h_out[sp] = issue_out(c - 1)
                if c - 1 + NSLOT < n_chunks:
                    h_g[sp] = issue_gather(c - 1 + NSLOT)
            if c >= 2:
                so = (c - 2) % NSLOT
                h_out[so].wait()
                h_out[so] = None
                if c - 2 + NSLOT < n_chunks:
                    h_x[so] = issue_x(c - 2 + NSLOT)

        for c in (n_chunks - 1,):
            s = c % NSLOT
            if h_add[s] is not None:
                h_add[s].wait()
                h_out[s] = issue_out(c)
        for h in h_out:
            if h is not None:
                h.wait()

    return k(x2, y2, w, idx_template)


def kernel(x, y, W_lookup):
    b, h, d = x.shape
    x2 = jnp.transpose(x, (1, 0, 2)).reshape(b * h, d)
    y2 = jnp.transpose(y).astype(jnp.int32)
    out2 = _lookup_add(x2, y2, W_lookup)
    return jnp.transpose(out2.reshape(h, b, d), (1, 0, 2))


# speedup vs baseline: 1.0639x; 1.0639x over previous
import functools

import jax
import jax.numpy as jnp
from jax import lax
from jax.experimental import pallas as pl
from jax.experimental.pallas import tpu as pltpu
from jax.experimental.pallas import tpu_sc as plsc

D = 128
LANES = 16
NC = 2
NS = 16
NW = NC * NS
CH = 128
RD = 64
RV = CH - RD
NSLOT = 3


def _lookup_add(x2, y2, w):
    n, _ = x2.shape
    hist, bsz = y2.shape
    n_chunks = hist
    b_w = bsz // NW
    mesh = plsc.VectorSubcoreMesh(core_axis_name="c", subcore_axis_name="s")

    idx_template = (
        jnp.arange(NSLOT * RD, dtype=jnp.int32).reshape(NSLOT, RD)
    )

    scratch = (
        [pltpu.VMEM((hist, b_w), jnp.int32)]
        + [pltpu.VMEM((NSLOT, RD), jnp.int32)]
        + [pltpu.VMEM((NSLOT, RD), jnp.int32)]
        + [pltpu.VMEM((CH, D), jnp.float32) for _ in range(NSLOT)]
        + [pltpu.VMEM((RV, D), jnp.float32) for _ in range(NSLOT)]
        + [pltpu.VMEM_SHARED((NS * NSLOT * RD, D), jnp.float32)]
        + [pltpu.SemaphoreType.DMA for _ in range(6 * NSLOT)]
    )

    @functools.partial(
        pl.kernel,
        mesh=mesh,
        out_type=jax.ShapeDtypeStruct(x2.shape, jnp.float32),
        scratch_types=scratch,
    )
    def k(x_hbm, y_hbm, w_hbm, idxt_hbm, out_hbm, idx_all, idxt, idxa,
          *bufs_and_sems):
        rows = bufs_and_sems[:NSLOT]
        xb_v = bufs_and_sems[NSLOT:2 * NSLOT]
        shared = bufs_and_sems[2 * NSLOT]
        sems = list(bufs_and_sems[2 * NSLOT + 1:])
        gsem = [sems.pop(0) for _ in range(NSLOT)]
        xdsem = [sems.pop(0) for _ in range(NSLOT)]
        xvsem = [sems.pop(0) for _ in range(NSLOT)]
        asem = [sems.pop(0) for _ in range(NSLOT)]
        odsem = [sems.pop(0) for _ in range(NSLOT)]
        ovsem = [sems.pop(0) for _ in range(NSLOT)]

        sid = lax.axis_index("s")
        wid = sid * NC + lax.axis_index("c")
        b_base = wid * b_w
        s_base = sid * (NSLOT * RD)

        pltpu.sync_copy(y_hbm.at[:, pl.ds(b_base, b_w)], idx_all)
        pltpu.sync_copy(idxt_hbm, idxt)
        for sl in range(NSLOT):
            for t in range(RD // LANES):
                dl = pl.ds(t * LANES, LANES)
                idxa[sl, dl] = idxt[sl, dl] + s_base

        def issue_gather(c):
            s = c % NSLOT
            return pltpu.async_copy(
                w_hbm.at[idx_all.at[c]],
                rows[s],
                gsem[s],
            )

        def issue_xd(c):
            s = c % NSLOT
            return pltpu.async_copy(
                x_hbm.at[pl.ds(c * bsz + b_base, RD)],
                shared.at[pl.ds(s_base + s * RD, RD)],
                xdsem[s],
            )

        def issue_xv(c):
            s = c % NSLOT
            return pltpu.async_copy(
                x_hbm.at[pl.ds(c * bsz + b_base + RD, RV)],
                xb_v[s],
                xvsem[s],
            )

        def issue_add(c):
            s = c % NSLOT
            return pltpu.async_copy(
                rows[s].at[pl.ds(0, RD)], shared.at[idxa.at[s]],
                asem[s], add=True,
            )

        def issue_od(c):
            s = c % NSLOT
            return pltpu.async_copy(
                shared.at[pl.ds(s_base + s * RD, RD)],
                out_hbm.at[pl.ds(c * bsz + b_base, RD)],
                odsem[s],
            )

        def issue_ov(c):
            s = c % NSLOT
            return pltpu.async_copy(
                xb_v[s],
                out_hbm.at[pl.ds(c * bsz + b_base + RD, RV)],
                ovsem[s],
            )

        h_g = [None] * NSLOT
        h_xd = [None] * NSLOT
        h_xv = [None] * NSLOT
        h_add = [None] * NSLOT
        h_od = [None] * NSLOT
        h_ov = [None] * NSLOT
        for c in range(min(NSLOT, n_chunks)):
            h_g[c] = issue_gather(c)
            h_xd[c] = issue_xd(c)
            h_xv[c] = issue_xv(c)

        for c in range(n_chunks):
            s = c % NSLOT
            h_g[s].wait()
            h_xd[s].wait()
            h_add[s] = issue_add(c)
            h_xv[s].wait()
            rows_c = rows[s]
            xb_c = xb_v[s]

            @plsc.parallel_loop(0, RV, 1)
            def add_body(ir):
                for t in range(D // LANES):
                    sl = pl.ds(t * LANES, LANES)
                    plsc.addupdate(xb_c.at[ir, sl], rows_c[RD + ir, sl])

            h_ov[s] = issue_ov(c)
            if c >= 1:
                sp = (c - 1) % NSLOT
                h_add[sp].wait()
                h_add[sp] = None
                h_od[sp] = issue_od(c - 1)
                if c + 2 < n_chunks:
                    h_g[sp] = issue_gather(c + 2)
            if c >= 2:
                so = (c - 2) % NSLOT
                h_od[so].wait()
                h_od[so] = None
                h_ov[so].wait()
                h_ov[so] = None
                if c + 1 < n_chunks:
                    h_xd[so] = issue_xd(c + 1)
                    h_xv[so] = issue_xv(c + 1)

        s = (n_chunks - 1) % NSLOT
        if h_add[s] is not None:
            h_add[s].wait()
            h_od[s] = issue_od(n_chunks - 1)
        for h in h_od + h_ov:
            if h is not None:
                h.wait()

    return k(x2, y2, w, idx_template)


def kernel(x, y, W_lookup):
    b, h, d = x.shape
    x2 = jnp.transpose(x, (1, 0, 2)).reshape(b * h, d)
    y2 = jnp.transpose(y).astype(jnp.int32)
    out2 = _lookup_add(x2, y2, W_lookup)
    return jnp.transpose(out2.reshape(h, b, d), (1, 0, 2))


# speedup vs baseline: 1.2156x; 1.1425x over previous
import functools

import jax
import jax.numpy as jnp
from jax import lax
from jax.experimental import pallas as pl
from jax.experimental.pallas import tpu as pltpu
from jax.experimental.pallas import tpu_sc as plsc

D = 128
LANES = 16
NC = 2
NS = 16
NW = NC * NS
CH = 128
NSLOT = 3


def _lookup_add(x2, y2, w):
    n, _ = x2.shape
    hist, bsz = y2.shape
    n_chunks = hist
    b_w = bsz // NW
    mesh = plsc.VectorSubcoreMesh(core_axis_name="c", subcore_axis_name="s")

    idx_template = (
        jnp.arange(NSLOT * CH, dtype=jnp.int32).reshape(NSLOT, CH)
    )

    scratch = (
        [pltpu.VMEM((hist, b_w), jnp.int32)]
        + [pltpu.VMEM((NSLOT, CH), jnp.int32)]
        + [pltpu.VMEM((NSLOT, CH), jnp.int32)]
        + [pltpu.VMEM((CH, D), jnp.float32) for _ in range(NSLOT)]
        + [pltpu.VMEM_SHARED((NS * NSLOT * CH, D), jnp.float32)]
        + [pltpu.SemaphoreType.DMA for _ in range(4 * NSLOT)]
    )

    @functools.partial(
        pl.kernel,
        mesh=mesh,
        out_type=jax.ShapeDtypeStruct(x2.shape, jnp.float32),
        scratch_types=scratch,
    )
    def k(x_hbm, y_hbm, w_hbm, idxt_hbm, out_hbm, idx_all, idxt, idxa,
          *bufs_and_sems):
        rows = bufs_and_sems[:NSLOT]
        shared = bufs_and_sems[NSLOT]
        gsem = bufs_and_sems[NSLOT + 1:2 * NSLOT + 1]
        xsem = bufs_and_sems[2 * NSLOT + 1:3 * NSLOT + 1]
        asem = bufs_and_sems[3 * NSLOT + 1:4 * NSLOT + 1]
        osem = bufs_and_sems[4 * NSLOT + 1:]

        sid = lax.axis_index("s")
        wid = sid * NC + lax.axis_index("c")
        b_base = wid * b_w
        s_base = sid * (NSLOT * CH)

        pltpu.sync_copy(y_hbm.at[:, pl.ds(b_base, b_w)], idx_all)
        pltpu.sync_copy(idxt_hbm, idxt)
        for sl in range(NSLOT):
            for t in range(CH // LANES):
                dl = pl.ds(t * LANES, LANES)
                idxa[sl, dl] = idxt[sl, dl] + s_base

        def issue_gather(c):
            s = c % NSLOT
            return pltpu.async_copy(
                w_hbm.at[idx_all.at[c]],
                rows[s],
                gsem[s],
            )

        def issue_x(c):
            s = c % NSLOT
            return pltpu.async_copy(
                x_hbm.at[pl.ds(c * bsz + b_base, CH)],
                shared.at[pl.ds(s_base + s * CH, CH)],
                xsem[s],
            )

        def issue_add(c):
            s = c % NSLOT
            return pltpu.async_copy(
                rows[s], shared.at[idxa.at[s]], asem[s], add=True
            )

        def issue_out(c):
            s = c % NSLOT
            return pltpu.async_copy(
                shared.at[pl.ds(s_base + s * CH, CH)],
                out_hbm.at[pl.ds(c * bsz + b_base, CH)],
                osem[s],
            )

        h_g = [None] * NSLOT
        h_x = [None] * NSLOT
        h_add = [None] * NSLOT
        h_out = [None] * NSLOT
        for c in range(min(NSLOT, n_chunks)):
            h_g[c] = issue_gather(c)
            h_x[c] = issue_x(c)

        for c in range(n_chunks):
            s = c % NSLOT
            h_g[s].wait()
            h_x[s].wait()
            h_add[s] = issue_add(c)
            if c >= 1:
                sp = (c - 1) % NSLOT
                h_add[sp].wait()
                h_add[sp] = None
                h_out[sp] = issue_out(c - 1)
                if c - 1 + NSLOT < n_chunks:
                    h_g[sp] = issue_gather(c - 1 + NSLOT)
            if c >= 2:
                so = (c - 2) % NSLOT
                h_out[so].wait()
                h_out[so] = None
                if c - 2 + NSLOT < n_chunks:
                    h_x[so] = issue_x(c - 2 + NSLOT)

        for c in (n_chunks - 1,):
            s = c % NSLOT
            if h_add[s] is not None:
                h_add[s].wait()
                h_out[s] = issue_out(c)
        for h in h_out:
            if h is not None:
                h.wait()

    return k(x2, y2, w, idx_template)


def kernel(x, y, W_lookup):
    b, h, d = x.shape
    x2 = jnp.transpose(x, (1, 0, 2)).reshape(b * h, d)
    y2 = jnp.transpose(y).astype(jnp.int32)
    out2 = _lookup_add(x2, y2, W_lookup)
    return jnp.transpose(out2.reshape(h, b, d), (1, 0, 2))
